# Initial kernel scaffold; baseline (speedup 1.0000x reference)
import jax
import jax.numpy as jnp
from jax.experimental import pallas as pl


def kernel(x, edge_index, W1, b1, W2, b2, W3, b3):
    raise NotImplementedError("write your pallas kernel here")


# speedup vs baseline: 21.7799x; 21.7799x over previous
import functools

import jax
import jax.numpy as jnp
from jax import lax
from jax.experimental import pallas as pl
from jax.experimental.pallas import tpu as pltpu
from jax.experimental.pallas import tpu_sc as plsc

N = 10000
E = 320000

NW = 32
NSUB = 16
CHUNK = 128
NCHUNK = 80
E_PAD = NW * NCHUNK * CHUNK
NP = 10112
RPT = NP // NSUB

_MESH = plsc.VectorSubcoreMesh(core_axis_name="c", subcore_axis_name="s")


def _zero_rows(zbuf, feat):
    def body(i, carry):
        for j in range(feat // 16):
            zbuf[i, pl.ds(j * 16, 16)] = jnp.zeros((16,), jnp.float32)
        return carry
    lax.fori_loop(0, RPT, body, 0)


def _make_edge_pass(feat):
    @functools.partial(
        pl.kernel,
        mesh=_MESH,
        out_type=jax.ShapeDtypeStruct((2, NP, feat), jnp.float32),
        compiler_params=pltpu.CompilerParams(use_tc_tiling_on_sc=False),
        scratch_types=[
            pltpu.VMEM((NCHUNK, CHUNK), jnp.int32),
            pltpu.VMEM((NCHUNK, CHUNK), jnp.int32),
            pltpu.VMEM((CHUNK, feat), jnp.float32),
            pltpu.VMEM((RPT, feat), jnp.float32),
            pltpu.SemaphoreType.DMA,
            pltpu.VMEM_SHARED((NP, feat), jnp.float32),
        ],
    )
    def edge_pass(y_hbm, src_hbm, dst_hbm, out_hbm, src_v, dst_v, rows_v, zbuf,
                  sem, acc_sh):
        c = lax.axis_index("c")
        s = lax.axis_index("s")
        wid = c * NSUB + s
        row0 = s * RPT

        _zero_rows(zbuf, feat)
        pltpu.sync_copy(zbuf, acc_sh.at[pl.ds(row0, RPT)])
        pltpu.sync_copy(src_hbm.at[pl.ds(wid * NCHUNK, NCHUNK)], src_v)
        pltpu.sync_copy(dst_hbm.at[pl.ds(wid * NCHUNK, NCHUNK)], dst_v)
        plsc.subcore_barrier()

        def step(j, carry):
            pltpu.async_copy(y_hbm.at[src_v.at[j]], rows_v, sem).wait()
            pltpu.sync_copy(rows_v, acc_sh.at[dst_v.at[j]], add=True)
            return carry
        lax.fori_loop(0, NCHUNK, step, 0)

        plsc.subcore_barrier()
        pltpu.sync_copy(acc_sh.at[pl.ds(row0, RPT)],
                        out_hbm.at[c].at[pl.ds(row0, RPT)])

    return edge_pass


_DEG_F = 16


@functools.partial(
    pl.kernel,
    mesh=_MESH,
    out_type=jax.ShapeDtypeStruct((2, NP, _DEG_F), jnp.float32),
    compiler_params=pltpu.CompilerParams(use_tc_tiling_on_sc=False),
    scratch_types=[
        pltpu.VMEM((NCHUNK, CHUNK), jnp.int32),
        pltpu.VMEM((CHUNK, _DEG_F), jnp.float32),
        pltpu.VMEM((RPT, _DEG_F), jnp.float32),
        pltpu.VMEM_SHARED((NP, _DEG_F), jnp.float32),
    ],
)
def _deg_pass(dst_hbm, out_hbm, dst_v, ones_v, zbuf, acc_sh):
    c = lax.axis_index("c")
    s = lax.axis_index("s")
    wid = c * NSUB + s
    row0 = s * RPT

    _zero_rows(zbuf, _DEG_F)

    def fill_ones(i, carry):
        ones_v[i, pl.ds(0, 16)] = jnp.ones((16,), jnp.float32)
        return carry
    lax.fori_loop(0, CHUNK, fill_ones, 0)

    pltpu.sync_copy(zbuf, acc_sh.at[pl.ds(row0, RPT)])
    pltpu.sync_copy(dst_hbm.at[pl.ds(wid * NCHUNK, NCHUNK)], dst_v)
    plsc.subcore_barrier()

    def step(j, carry):
        pltpu.sync_copy(ones_v, acc_sh.at[dst_v.at[j]], add=True)
        return carry
    lax.fori_loop(0, NCHUNK, step, 0)

    plsc.subcore_barrier()
    pltpu.sync_copy(acc_sh.at[pl.ds(row0, RPT)],
                    out_hbm.at[c].at[pl.ds(row0, RPT)])


_edge32 = _make_edge_pass(32)
_edge16 = _make_edge_pass(16)


def _k1_body(deg_ref, x_ref, w_ref, y_ref, dinv_ref):
    deg = deg_ref[0, :N, 0:1] + deg_ref[1, :N, 0:1] + 1.0
    dinv = lax.rsqrt(deg)
    dinv_ref[...] = dinv
    y_ref[...] = jnp.dot(x_ref[...], w_ref[...],
                         preferred_element_type=jnp.float32) * dinv


def _k1(degs, x, W1):
    return pl.pallas_call(
        _k1_body,
        out_shape=(jax.ShapeDtypeStruct((N, 32), jnp.float32),
                   jax.ShapeDtypeStruct((N, 1), jnp.float32)),
    )(degs, x, W1)


def _kmid_body(acc_ref, y_ref, dinv_ref, b_ref, w_ref, o_ref):
    dinv = dinv_ref[...]
    h = (acc_ref[0, :N, :] + acc_ref[1, :N, :] + y_ref[...]) * dinv + b_ref[...]
    h = jnp.maximum(h, 0.0)
    o_ref[...] = jnp.dot(h, w_ref[...], preferred_element_type=jnp.float32) * dinv


def _kmid(acc, y, dinv, b, W, f_out):
    return pl.pallas_call(
        _kmid_body,
        out_shape=jax.ShapeDtypeStruct((N, f_out), jnp.float32),
    )(acc, y, dinv, b, W)


def _k4_body(acc_ref, y_ref, dinv_ref, b_ref, o_ref):
    full = (acc_ref[0, :N, :] + acc_ref[1, :N, :] + y_ref[...]) * dinv_ref[...]
    o_ref[...] = full[:, :8] + b_ref[...]


def _k4(acc, y, dinv, b):
    return pl.pallas_call(
        _k4_body,
        out_shape=jax.ShapeDtypeStruct((N, 8), jnp.float32),
    )(acc, y, dinv, b)


def kernel(x, edge_index, W1, b1, W2, b2, W3, b3):
    src = edge_index[0]
    dst = edge_index[1]
    pad = E_PAD - E
    src_p = jnp.concatenate(
        [src, jnp.zeros((pad,), jnp.int32)]).reshape(NW * NCHUNK, CHUNK)
    dst_p = jnp.concatenate(
        [dst, jnp.full((pad,), N, jnp.int32)]).reshape(NW * NCHUNK, CHUNK)

    b1r = b1.reshape(1, 32)
    b2r = b2.reshape(1, 16)
    b3r = b3.reshape(1, 8)
    W3p = jnp.pad(W3, ((0, 0), (0, 8)))

    degs = _deg_pass(dst_p)
    y1, dinv = _k1(degs, x, W1)
    acc1 = _edge32(y1, src_p, dst_p)
    y2 = _kmid(acc1, y1, dinv, b1r, W2, 16)
    acc2 = _edge16(y2, src_p, dst_p)
    y3 = _kmid(acc2, y2, dinv, b2r, W3p, 16)
    acc3 = _edge16(y3, src_p, dst_p)
    return _k4(acc3, y3, dinv, b3r)
